# baseline (speedup 1.0000x reference)
import functools

import jax
import jax.numpy as jnp
from jax import lax
from jax.experimental import pallas as pl
from jax.experimental.pallas import tpu as pltpu
from jax.experimental.pallas import tpu_sc as plsc

VOCAB = 100000
H = 128
NP = 16
B = 1024
L = 200

NC = 2
NS = 16
NW = NC * NS
ROWS_PER_W = B // NW
CH = 40
NCH = L // CH
TCH = ROWS_PER_W * NCH
OUT_L = NP + L
NBUF = 8


def _soft_embedding_body(tokens_hbm, table_hbm, prompt_hbm, out_hbm,
                         idx_v, rows_v, prompt_v, *sems):
    gsems = sems[:NBUF]
    wsems = sems[NBUF:2 * NBUF]
    psem = sems[2 * NBUF]
    wid = lax.axis_index("s") * NC + lax.axis_index("c")
    first_row = wid * ROWS_PER_W

    pltpu.sync_copy(tokens_hbm.at[pl.ds(wid * TCH, TCH)], idx_v)
    pltpu.sync_copy(prompt_hbm, prompt_v)

    def do_block(c0, first):
        descs = []
        for k in range(NBUF):
            c = c0 + k
            if not first:
                pltpu.make_async_copy(
                    rows_v.at[k], out_hbm.at[pl.ds(0, CH)], wsems[k]).wait()
            descs.append(pltpu.async_copy(
                table_hbm.at[idx_v.at[c]], rows_v.at[k], gsems[k]))

            @pl.when(c % NCH == 0)
            def _():
                pltpu.async_copy(
                    prompt_v,
                    out_hbm.at[pl.ds((first_row + c // NCH) * OUT_L, NP)],
                    psem)

        for k in range(NBUF):
            c = c0 + k
            descs[k].wait()
            b = first_row + c // NCH
            base = b * OUT_L + NP + (c % NCH) * CH
            pltpu.async_copy(
                rows_v.at[k], out_hbm.at[pl.ds(base, CH)], wsems[k])

    do_block(0, True)

    @pl.loop(NBUF, TCH, step=NBUF)
    def _block(c0):
        do_block(c0, False)

    for k in range(NBUF):
        pltpu.make_async_copy(
            rows_v.at[k], out_hbm.at[pl.ds(0, CH)], wsems[k]).wait()

    @pl.loop(0, ROWS_PER_W)
    def _drain_prompt(i):
        pltpu.make_async_copy(
            prompt_v, out_hbm.at[pl.ds(0, NP)], psem).wait()


@jax.jit
def _soft_embedding(tokens2, table, prompt_embeds):
    mesh = plsc.VectorSubcoreMesh(
        core_axis_name="c", subcore_axis_name="s",
        num_cores=NC, num_subcores=NS)
    flat = pl.kernel(
        _soft_embedding_body,
        out_type=jax.ShapeDtypeStruct((B * OUT_L, H), jnp.float32),
        mesh=mesh,
        scratch_types=(
            [pltpu.VMEM((TCH, CH), jnp.int32),
             pltpu.VMEM((NBUF, CH, H), jnp.float32),
             pltpu.VMEM((NP, H), jnp.float32)]
            + [pltpu.SemaphoreType.DMA] * (2 * NBUF + 1)
        ),
    )(tokens2, table, prompt_embeds)
    return flat.reshape(B, OUT_L, H)


def kernel(tokens, table, prompt_embeds):
    tokens2 = tokens.astype(jnp.int32).reshape(B * L // CH, CH)
    return _soft_embedding(tokens2, table, prompt_embeds)
